# baseline (speedup 1.0000x reference)
import functools

import jax
import jax.numpy as jnp
from jax import lax
from jax.experimental import pallas as pl
from jax.experimental.pallas import tpu as pltpu
from jax.experimental.pallas import tpu_sc as plsc

D_MODEL = 64
SCALE = 8.0
NUM_CORES = 2
NUM_SUBCORES = 16
NUM_WORKERS = NUM_CORES * NUM_SUBCORES
VOCAB = 1000000
NPAIR = VOCAB // 2
NBLK = VOCAB // 128
TOTAL_K = 248
SEQ = 200
NBUF = 4

_MESH = plsc.VectorSubcoreMesh(core_axis_name="c", subcore_axis_name="s")
_PARAMS = pltpu.CompilerParams(
    use_tc_tiling_on_sc=True, needs_layout_passes=False
)


def _worker_id():
    return lax.axis_index("s") * NUM_CORES + lax.axis_index("c")


def _bc16(v):
    return jnp.broadcast_to(v, (16,))


@functools.partial(
    pl.kernel,
    mesh=_MESH,
    out_type=jax.ShapeDtypeStruct((NPAIR, 128), jnp.float32),
    compiler_params=_PARAMS,
    scratch_types=(
        [pltpu.VMEM((64, 128), jnp.float32) for _ in range(NBUF)]
        + [pltpu.VMEM((64, 128), jnp.float32) for _ in range(NBUF)]
        + [pltpu.SemaphoreType.DMA for _ in range(2 * NBUF)]
    ),
)
def _pack_kernel(tt_hbm, packed_hbm, *scratch):
    wid = _worker_id()
    bufs = scratch[0:NBUF]
    bufts = scratch[NBUF:2 * NBUF]
    isem = scratch[2 * NBUF:3 * NBUF]
    osem = scratch[3 * NBUF:4 * NBUF]
    iota16 = lax.iota(jnp.int32, 16)

    def blk(k):
        return jnp.minimum(wid + k * NUM_WORKERS, NBLK - 1)

    def start_in(k, slot):
        pltpu.async_copy(
            tt_hbm.at[:, pl.ds(blk(k) * 128, 128)], bufs[slot], isem[slot]
        )

    def start_out(k, slot):
        pltpu.async_copy(
            bufts[slot], packed_hbm.at[pl.ds(blk(k) * 64, 64)], osem[slot]
        )

    def wait_in(slot):
        pltpu.make_async_copy(
            tt_hbm.at[:, pl.ds(0, 128)], bufs[slot], isem[slot]
        ).wait()

    def wait_out(slot):
        pltpu.make_async_copy(
            bufts[slot], packed_hbm.at[pl.ds(0, 64)], osem[slot]
        ).wait()

    halfvec = (iota16 & 1) * 64

    def transpose_block(slot, n_pairs):
        buf = bufs[slot]
        buft = bufts[slot]

        perms = [(iota16 + k) & 15 for k in range(16)]
        c2bases = [halfvec + perms[k] for k in range(16)]

        def tbody(t, carry):
            j0 = (t // 4) * 16
            d0v = _bc16((t % 4) * 16)
            colj = iota16 + j0
            pv = colj >> 1
            for k in range(16):
                rowd = d0v + perms[k]
                c2v = c2bases[k] + d0v
                val = plsc.load_gather(buf, [rowd, colj]) * SCALE
                plsc.store_scatter(buft, [pv, c2v], val)
            return carry

        lax.fori_loop(0, (2 * n_pairs // 16) * 4, tbody, 0)

    def half(k, slot, first):
        wait_in(slot)
        if not first:
            wait_out(slot)
        transpose_block(slot, 64)
        start_out(k, slot)

    for k in range(NBUF):
        start_in(k, k)
    for k in range(NBUF):
        half(k, k, True)
        start_in(k + NBUF, k)

    def ring(q, carry):
        for slot in range(NBUF):
            k = NBUF * q + slot
            half(k, slot, False)

            @pl.when(k + NBUF < TOTAL_K)
            def _():
                start_in(k + NBUF, slot)

        return carry

    lax.fori_loop(1, TOTAL_K // NBUF, ring, 0)
    for slot in range(NBUF):
        wait_out(slot)

    @pl.when(wid == NUM_WORKERS - 1)
    def _():
        for d in range(64):
            pltpu.sync_copy(
                tt_hbm.at[d, pl.ds(NBLK * 128, 64)],
                bufs[0].at[d, pl.ds(0, 64)],
            )
        transpose_block(0, 32)
        pltpu.sync_copy(
            bufts[0].at[pl.ds(0, 32)], packed_hbm.at[pl.ds(NBLK * 64, 32)]
        )


@functools.partial(
    pl.kernel,
    mesh=_MESH,
    out_type=jax.ShapeDtypeStruct((SEQ, 64, 4096), jnp.float32),
    compiler_params=_PARAMS,
    scratch_types=(
        [pltpu.VMEM((SEQ, 128), jnp.int32)]
        + [pltpu.VMEM((128,), jnp.int32) for _ in range(NBUF)]
        + [pltpu.VMEM((128, 128), jnp.float32) for _ in range(NBUF)]
        + [pltpu.VMEM((64, 128), jnp.float32) for _ in range(2)]
        + [pltpu.SemaphoreType.DMA for _ in range(NBUF + 2)]
    ),
)
def _gather_kernel(packed_hbm, xt_hbm, out_hbm, xb, *scratch):
    wid = _worker_id()
    bcol = wid * 128
    pidxs = scratch[0:NBUF]
    rows = scratch[NBUF:2 * NBUF]
    bufts = scratch[2 * NBUF:2 * NBUF + 2]
    gsem = scratch[2 * NBUF + 2:3 * NBUF + 2]
    osem = scratch[3 * NBUF + 2:3 * NBUF + 4]
    iota16 = lax.iota(jnp.int32, 16)
    perms = [(iota16 + k) & 15 for k in range(16)]
    pltpu.sync_copy(xt_hbm.at[:, pl.ds(bcol, 128)], xb)

    def start_gather(s, slot):
        pidx = pidxs[slot]
        for g in range(8):
            sl = pl.ds(g * 16, 16)
            pidx[sl] = xb[s, sl] >> 1
        pltpu.async_copy(packed_hbm.at[pidx], rows[slot], gsem[slot])

    def wait_gather(slot):
        pltpu.make_async_copy(
            packed_hbm.at[pl.ds(0, 128)], rows[slot], gsem[slot]
        ).wait()

    def start_out(s, oslot):
        pltpu.async_copy(
            bufts[oslot], out_hbm.at[s, :, pl.ds(bcol, 128)], osem[oslot]
        )

    def wait_out(oslot):
        pltpu.make_async_copy(
            bufts[oslot], out_hbm.at[0, :, pl.ds(bcol, 128)], osem[oslot]
        ).wait()

    def transpose(s, slot, oslot):
        rv = rows[slot]
        buft = bufts[oslot]
        def tbody(t, carry):
            j0 = (t // 4) * 16
            d0v = _bc16((t % 4) * 16)
            jv = iota16 + j0
            hd = (xb[s, pl.ds(j0, 16)] & 1) * 64 + d0v
            for k in range(16):
                dvec = d0v + perms[k]
                val = plsc.load_gather(rv, [jv, hd + perms[k]])
                plsc.store_scatter(buft, [dvec, jv], val)
            return carry

        lax.fori_loop(0, 32, tbody, 0)

    def half(s, slot, first):
        oslot = slot % 2
        wait_gather(slot)
        if not first:
            wait_out(oslot)
        transpose(s, slot, oslot)
        start_out(s, oslot)

    for s in range(NBUF):
        start_gather(s, s)
    for s in range(2):
        half(s, s, True)
        start_gather(s + NBUF, s)
    for s in range(2, NBUF):
        half(s, s, False)
        start_gather(s + NBUF, s)

    def ring(q, carry):
        for slot in range(NBUF):
            s = NBUF * q + slot
            half(s, slot, False)

            @pl.when(s + NBUF < SEQ)
            def _():
                start_gather(s + NBUF, slot)

        return carry

    lax.fori_loop(1, SEQ // NBUF, ring, 0)
    wait_out(0)
    wait_out(1)


def kernel(x, table):
    packed = _pack_kernel(table.T)
    outp = _gather_kernel(packed, x.T)
    return jnp.transpose(outp, (2, 0, 1))


# speedup vs baseline: 1.0006x; 1.0006x over previous
import functools

import jax
import jax.numpy as jnp
from jax import lax
from jax.experimental import pallas as pl
from jax.experimental.pallas import tpu as pltpu
from jax.experimental.pallas import tpu_sc as plsc

D_MODEL = 64
SCALE = 8.0
NUM_CORES = 2
NUM_SUBCORES = 16
NUM_WORKERS = NUM_CORES * NUM_SUBCORES
VOCAB = 1000000
NPAIR = VOCAB // 2
NBLK = VOCAB // 128
TOTAL_K = 248
SEQ = 200
NBUF = 4

_MESH = plsc.VectorSubcoreMesh(core_axis_name="c", subcore_axis_name="s")
_PARAMS = pltpu.CompilerParams(
    use_tc_tiling_on_sc=True,
    needs_layout_passes=False,
    disable_bounds_checks=True,
)


def _worker_id():
    return lax.axis_index("s") * NUM_CORES + lax.axis_index("c")


def _bc16(v):
    return jnp.broadcast_to(v, (16,))


@functools.partial(
    pl.kernel,
    mesh=_MESH,
    out_type=jax.ShapeDtypeStruct((NPAIR, 128), jnp.float32),
    compiler_params=_PARAMS,
    scratch_types=(
        [pltpu.VMEM((64, 128), jnp.float32) for _ in range(NBUF)]
        + [pltpu.VMEM((64, 128), jnp.float32) for _ in range(NBUF)]
        + [pltpu.SemaphoreType.DMA for _ in range(2 * NBUF)]
    ),
)
def _pack_kernel(tt_hbm, packed_hbm, *scratch):
    wid = _worker_id()
    bufs = scratch[0:NBUF]
    bufts = scratch[NBUF:2 * NBUF]
    isem = scratch[2 * NBUF:3 * NBUF]
    osem = scratch[3 * NBUF:4 * NBUF]
    iota16 = lax.iota(jnp.int32, 16)

    def blk(k):
        return jnp.minimum(wid + k * NUM_WORKERS, NBLK - 1)

    def start_in(k, slot):
        pltpu.async_copy(
            tt_hbm.at[:, pl.ds(blk(k) * 128, 128)], bufs[slot], isem[slot]
        )

    def start_out(k, slot):
        pltpu.async_copy(
            bufts[slot], packed_hbm.at[pl.ds(blk(k) * 64, 64)], osem[slot]
        )

    def wait_in(slot):
        pltpu.make_async_copy(
            tt_hbm.at[:, pl.ds(0, 128)], bufs[slot], isem[slot]
        ).wait()

    def wait_out(slot):
        pltpu.make_async_copy(
            bufts[slot], packed_hbm.at[pl.ds(0, 64)], osem[slot]
        ).wait()

    halfvec = (iota16 & 1) * 64

    def transpose_block(slot, n_pairs):
        buf = bufs[slot]
        buft = bufts[slot]

        perms = [(iota16 + k) & 15 for k in range(16)]
        c2bases = [halfvec + perms[k] for k in range(16)]

        def tbody(t, carry):
            j0 = (t // 4) * 16
            d0v = _bc16((t % 4) * 16)
            colj = iota16 + j0
            pv = colj >> 1
            for k in range(16):
                rowd = d0v + perms[k]
                c2v = c2bases[k] + d0v
                val = plsc.load_gather(buf, [rowd, colj]) * SCALE
                plsc.store_scatter(buft, [pv, c2v], val)
            return carry

        lax.fori_loop(0, (2 * n_pairs // 16) * 4, tbody, 0)

    def half(k, slot, first):
        wait_in(slot)
        if not first:
            wait_out(slot)
        transpose_block(slot, 64)
        start_out(k, slot)

    for k in range(NBUF):
        start_in(k, k)
    for k in range(NBUF):
        half(k, k, True)
        start_in(k + NBUF, k)

    def ring(q, carry):
        for slot in range(NBUF):
            k = NBUF * q + slot
            half(k, slot, False)

            @pl.when(k + NBUF < TOTAL_K)
            def _():
                start_in(k + NBUF, slot)

        return carry

    lax.fori_loop(1, TOTAL_K // NBUF, ring, 0)
    for slot in range(NBUF):
        wait_out(slot)

    @pl.when(wid == NUM_WORKERS - 1)
    def _():
        for d in range(64):
            pltpu.sync_copy(
                tt_hbm.at[d, pl.ds(NBLK * 128, 64)],
                bufs[0].at[d, pl.ds(0, 64)],
            )
        transpose_block(0, 32)
        pltpu.sync_copy(
            bufts[0].at[pl.ds(0, 32)], packed_hbm.at[pl.ds(NBLK * 64, 32)]
        )


@functools.partial(
    pl.kernel,
    mesh=_MESH,
    out_type=jax.ShapeDtypeStruct((SEQ, 64, 4096), jnp.float32),
    compiler_params=_PARAMS,
    scratch_types=(
        [pltpu.VMEM((SEQ, 128), jnp.int32)]
        + [pltpu.VMEM((128,), jnp.int32) for _ in range(NBUF)]
        + [pltpu.VMEM((128, 128), jnp.float32) for _ in range(NBUF)]
        + [pltpu.VMEM((64, 128), jnp.float32) for _ in range(2)]
        + [pltpu.SemaphoreType.DMA for _ in range(NBUF + 2)]
    ),
)
def _gather_kernel(packed_hbm, xt_hbm, out_hbm, xb, *scratch):
    wid = _worker_id()
    bcol = wid * 128
    pidxs = scratch[0:NBUF]
    rows = scratch[NBUF:2 * NBUF]
    bufts = scratch[2 * NBUF:2 * NBUF + 2]
    gsem = scratch[2 * NBUF + 2:3 * NBUF + 2]
    osem = scratch[3 * NBUF + 2:3 * NBUF + 4]
    iota16 = lax.iota(jnp.int32, 16)
    perms = [(iota16 + k) & 15 for k in range(16)]
    pltpu.sync_copy(xt_hbm.at[:, pl.ds(bcol, 128)], xb)

    def start_gather(s, slot):
        pidx = pidxs[slot]
        for g in range(8):
            sl = pl.ds(g * 16, 16)
            pidx[sl] = xb[s, sl] >> 1
        pltpu.async_copy(packed_hbm.at[pidx], rows[slot], gsem[slot])

    def wait_gather(slot):
        pltpu.make_async_copy(
            packed_hbm.at[pl.ds(0, 128)], rows[slot], gsem[slot]
        ).wait()

    def start_out(s, oslot):
        pltpu.async_copy(
            bufts[oslot], out_hbm.at[s, :, pl.ds(bcol, 128)], osem[oslot]
        )

    def wait_out(oslot):
        pltpu.make_async_copy(
            bufts[oslot], out_hbm.at[0, :, pl.ds(bcol, 128)], osem[oslot]
        ).wait()

    def transpose(s, slot, oslot):
        rv = rows[slot]
        buft = bufts[oslot]
        def tbody(t, carry):
            j0 = (t // 4) * 16
            d0v = _bc16((t % 4) * 16)
            jv = iota16 + j0
            hd = (xb[s, pl.ds(j0, 16)] & 1) * 64 + d0v
            for k in range(16):
                dvec = d0v + perms[k]
                val = plsc.load_gather(rv, [jv, hd + perms[k]])
                plsc.store_scatter(buft, [dvec, jv], val)
            return carry

        lax.fori_loop(0, 32, tbody, 0)

    def half(s, slot, first):
        oslot = slot % 2
        wait_gather(slot)
        if not first:
            wait_out(oslot)
        transpose(s, slot, oslot)
        start_out(s, oslot)

    for s in range(NBUF):
        start_gather(s, s)
    for s in range(2):
        half(s, s, True)
        start_gather(s + NBUF, s)
    for s in range(2, NBUF):
        half(s, s, False)
        start_gather(s + NBUF, s)

    def ring(q, carry):
        for slot in range(NBUF):
            s = NBUF * q + slot
            half(s, slot, False)

            @pl.when(s + NBUF < SEQ)
            def _():
                start_gather(s + NBUF, slot)

        return carry

    lax.fori_loop(1, SEQ // NBUF, ring, 0)
    wait_out(0)
    wait_out(1)


def kernel(x, table):
    packed = _pack_kernel(table.T)
    outp = _gather_kernel(packed, x.T)
    return jnp.transpose(outp, (2, 0, 1))


# speedup vs baseline: 2.5213x; 2.5199x over previous
import functools

import jax
import jax.numpy as jnp
from jax import lax
from jax.experimental import pallas as pl
from jax.experimental.pallas import tpu as pltpu
from jax.experimental.pallas import tpu_sc as plsc

D_MODEL = 64
SCALE = 8.0
NUM_CORES = 2
NUM_SUBCORES = 16
NUM_WORKERS = NUM_CORES * NUM_SUBCORES
VOCAB = 1000000
NPAIR = VOCAB // 2
NBLK = VOCAB // 128
TOTAL_K = 248
SEQ = 200
NBUF = 4

_MESH = plsc.VectorSubcoreMesh(core_axis_name="c", subcore_axis_name="s")
_PARAMS = pltpu.CompilerParams(
    use_tc_tiling_on_sc=True,
    needs_layout_passes=False,
    disable_bounds_checks=True,
)


def _worker_id():
    return lax.axis_index("s") * NUM_CORES + lax.axis_index("c")


def _bc16(v):
    return jnp.broadcast_to(v, (16,))


@functools.partial(
    pl.kernel,
    mesh=_MESH,
    out_type=jax.ShapeDtypeStruct((NPAIR, 128), jnp.float32),
    compiler_params=_PARAMS,
    scratch_types=(
        [pltpu.VMEM((64, 128), jnp.float32) for _ in range(NBUF)]
        + [pltpu.VMEM((64, 128), jnp.float32) for _ in range(NBUF)]
        + [pltpu.SemaphoreType.DMA for _ in range(2 * NBUF)]
    ),
)
def _pack_kernel(tt_hbm, packed_hbm, *scratch):
    wid = _worker_id()
    bufs = scratch[0:NBUF]
    bufts = scratch[NBUF:2 * NBUF]
    isem = scratch[2 * NBUF:3 * NBUF]
    osem = scratch[3 * NBUF:4 * NBUF]
    iota16 = lax.iota(jnp.int32, 16)

    def blk(k):
        return jnp.minimum(wid + k * NUM_WORKERS, NBLK - 1)

    def start_in(k, slot):
        pltpu.async_copy(
            tt_hbm.at[:, pl.ds(blk(k) * 128, 128)], bufs[slot], isem[slot]
        )

    def start_out(k, slot):
        pltpu.async_copy(
            bufts[slot], packed_hbm.at[pl.ds(blk(k) * 64, 64)], osem[slot]
        )

    def wait_in(slot):
        pltpu.make_async_copy(
            tt_hbm.at[:, pl.ds(0, 128)], bufs[slot], isem[slot]
        ).wait()

    def wait_out(slot):
        pltpu.make_async_copy(
            bufts[slot], packed_hbm.at[pl.ds(0, 64)], osem[slot]
        ).wait()

    halfvec = (iota16 & 1) * 64

    def transpose_block(slot, n_pairs):
        buf = bufs[slot]
        buft = bufts[slot]

        perms = [(iota16 + k) & 15 for k in range(16)]
        c2bases = [halfvec + perms[k] for k in range(16)]

        def tbody(t, carry):
            j0 = (t // 4) * 16
            d0v = _bc16((t % 4) * 16)
            colj = iota16 + j0
            pv = colj >> 1
            vals = [
                plsc.load_gather(buf, [d0v + perms[k], colj]) * SCALE
                for k in range(16)
            ]
            for k in range(16):
                plsc.store_scatter(buft, [pv, c2bases[k] + d0v], vals[k])
            return carry

        lax.fori_loop(0, (2 * n_pairs // 16) * 4, tbody, 0)

    def half(k, slot, first):
        wait_in(slot)
        if not first:
            wait_out(slot)
        transpose_block(slot, 64)
        start_out(k, slot)

    for k in range(NBUF):
        start_in(k, k)
    for k in range(NBUF):
        half(k, k, True)
        start_in(k + NBUF, k)

    def ring(q, carry):
        for slot in range(NBUF):
            k = NBUF * q + slot
            half(k, slot, False)

            @pl.when(k + NBUF < TOTAL_K)
            def _():
                start_in(k + NBUF, slot)

        return carry

    lax.fori_loop(1, TOTAL_K // NBUF, ring, 0)
    for slot in range(NBUF):
        wait_out(slot)

    @pl.when(wid == NUM_WORKERS - 1)
    def _():
        for d in range(64):
            pltpu.sync_copy(
                tt_hbm.at[d, pl.ds(NBLK * 128, 64)],
                bufs[0].at[d, pl.ds(0, 64)],
            )
        transpose_block(0, 32)
        pltpu.sync_copy(
            bufts[0].at[pl.ds(0, 32)], packed_hbm.at[pl.ds(NBLK * 64, 32)]
        )


@functools.partial(
    pl.kernel,
    mesh=_MESH,
    out_type=jax.ShapeDtypeStruct((SEQ, 64, 4096), jnp.float32),
    compiler_params=_PARAMS,
    scratch_types=(
        [pltpu.VMEM((SEQ, 128), jnp.int32)]
        + [pltpu.VMEM((128,), jnp.int32) for _ in range(NBUF)]
        + [pltpu.VMEM((128, 128), jnp.float32) for _ in range(NBUF)]
        + [pltpu.VMEM((64, 128), jnp.float32) for _ in range(2)]
        + [pltpu.SemaphoreType.DMA for _ in range(NBUF + 2)]
    ),
)
def _gather_kernel(packed_hbm, xt_hbm, out_hbm, xb, *scratch):
    wid = _worker_id()
    bcol = wid * 128
    pidxs = scratch[0:NBUF]
    rows = scratch[NBUF:2 * NBUF]
    bufts = scratch[2 * NBUF:2 * NBUF + 2]
    gsem = scratch[2 * NBUF + 2:3 * NBUF + 2]
    osem = scratch[3 * NBUF + 2:3 * NBUF + 4]
    iota16 = lax.iota(jnp.int32, 16)
    perms = [(iota16 + k) & 15 for k in range(16)]
    pltpu.sync_copy(xt_hbm.at[:, pl.ds(bcol, 128)], xb)

    def start_gather(s, slot):
        pidx = pidxs[slot]
        for g in range(8):
            sl = pl.ds(g * 16, 16)
            pidx[sl] = xb[s, sl] >> 1
        pltpu.async_copy(packed_hbm.at[pidx], rows[slot], gsem[slot])

    def wait_gather(slot):
        pltpu.make_async_copy(
            packed_hbm.at[pl.ds(0, 128)], rows[slot], gsem[slot]
        ).wait()

    def start_out(s, oslot):
        pltpu.async_copy(
            bufts[oslot], out_hbm.at[s, :, pl.ds(bcol, 128)], osem[oslot]
        )

    def wait_out(oslot):
        pltpu.make_async_copy(
            bufts[oslot], out_hbm.at[0, :, pl.ds(bcol, 128)], osem[oslot]
        ).wait()

    def transpose(s, slot, oslot):
        rv = rows[slot]
        buft = bufts[oslot]
        def tbody(t, carry):
            j0 = (t // 4) * 16
            d0v = _bc16((t % 4) * 16)
            jv = iota16 + j0
            hd = (xb[s, pl.ds(j0, 16)] & 1) * 64 + d0v
            vals = [
                plsc.load_gather(rv, [jv, hd + perms[k]])
                for k in range(16)
            ]
            for k in range(16):
                plsc.store_scatter(buft, [d0v + perms[k], jv], vals[k])
            return carry

        lax.fori_loop(0, 32, tbody, 0)

    def half(s, slot, first):
        oslot = slot % 2
        wait_gather(slot)
        if not first:
            wait_out(oslot)
        transpose(s, slot, oslot)
        start_out(s, oslot)

    for s in range(NBUF):
        start_gather(s, s)
    for s in range(2):
        half(s, s, True)
        start_gather(s + NBUF, s)
    for s in range(2, NBUF):
        half(s, s, False)
        start_gather(s + NBUF, s)

    def ring(q, carry):
        for slot in range(NBUF):
            s = NBUF * q + slot
            half(s, slot, False)

            @pl.when(s + NBUF < SEQ)
            def _():
                start_gather(s + NBUF, slot)

        return carry

    lax.fori_loop(1, SEQ // NBUF, ring, 0)
    wait_out(0)
    wait_out(1)


def kernel(x, table):
    packed = _pack_kernel(table.T)
    outp = _gather_kernel(packed, x.T)
    return jnp.transpose(outp, (2, 0, 1))
